# Initial kernel scaffold; baseline (speedup 1.0000x reference)
import jax
import jax.numpy as jnp
from jax import lax
from jax.experimental import pallas as pl
from jax.experimental.pallas import tpu as pltpu
from jax.experimental.pallas import tpu_sc as plsc

_N = 10000
_E = 160000
_NTILES = 16
_NCORES = 2
_EPT = _E // _NTILES
_CHUNK = 80
_NCHUNK = _EPT // _CHUNK
_RPT = _N // _NTILES
_ZROWS = 125
_NZ = _RPT // _ZROWS
_DW = 16

_mesh = plsc.VectorSubcoreMesh(
    core_axis_name="c", subcore_axis_name="s",
    num_cores=_NCORES, num_subcores=_NTILES)

_f32 = jnp.float32


def _zero_acc(s, zrows, zdeg, stage_v, zdeg_v, acc_sh, deg_sh):
    r0 = s * _RPT
    pltpu.sync_copy(zrows, stage_v)
    for i in range(_NZ):
        pltpu.sync_copy(stage_v, acc_sh.at[pl.ds(r0 + i * _ZROWS, _ZROWS)])
    if zdeg is not None:
        pltpu.sync_copy(zdeg, zdeg_v)
        pltpu.sync_copy(zdeg_v, deg_sh.at[pl.ds(r0, _RPT)])


def _edge_loop(table, src_v, dst_v, rows_v, acc_sh, ones_v, dstd_v, deg_sh):
    def body(j, carry):
        pltpu.sync_copy(table.at[src_v.at[j]], rows_v)
        pltpu.sync_copy(rows_v, acc_sh.at[dst_v.at[j]], add=True)
        if ones_v is not None:
            pltpu.sync_copy(ones_v, deg_sh.at[dstd_v.at[j]], add=True)
        return carry
    lax.fori_loop(0, _NCHUNK, body, 0)


def _flush(s, acc_sh, stage_v, out):
    r0 = s * _RPT
    for i in range(_NZ):
        pltpu.sync_copy(acc_sh.at[pl.ds(r0 + i * _ZROWS, _ZROWS)], stage_v)
        pltpu.sync_copy(stage_v, out.at[pl.ds(r0 + i * _ZROWS, _ZROWS)])


def _flush_deg(s, deg_sh, zdeg_v, out):
    r0 = s * _RPT
    pltpu.sync_copy(deg_sh.at[pl.ds(r0, _RPT)], zdeg_v)
    pltpu.sync_copy(zdeg_v, out.at[pl.ds(r0, _RPT)])


def _sc_layer1_body(xlo, xhi, src3, dst3, dst3b, zrows, zdeg, ones,
                    s1lo_out, s1hi_out, deg_out, deg2_out,
                    src_v, dst_v, dstd_v, rows_v, ones_v, stage_v, zdeg_v,
                    acc_sh, deg_sh):
    c = lax.axis_index("c")
    s = lax.axis_index("s")
    _zero_acc(s, zrows, zdeg, stage_v, zdeg_v, acc_sh, deg_sh)
    plsc.subcore_barrier()
    pltpu.sync_copy(ones, ones_v)
    pltpu.sync_copy(src3.at[s], src_v)
    pltpu.sync_copy(dst3.at[s], dst_v)

    @pl.when(c == 0)
    def _():
        _edge_loop(xlo, src_v, dst_v, rows_v, acc_sh, ones_v, dst_v, deg_sh)

    @pl.when(c == 1)
    def _():
        pltpu.sync_copy(dst3b.at[s], dstd_v)
        _edge_loop(xhi, src_v, dst_v, rows_v, acc_sh, ones_v, dstd_v, deg_sh)

    plsc.subcore_barrier()

    @pl.when(c == 0)
    def _():
        _flush(s, acc_sh, stage_v, s1lo_out)
        _flush_deg(s, deg_sh, zdeg_v, deg_out)

    @pl.when(c == 1)
    def _():
        _flush(s, acc_sh, stage_v, s1hi_out)
        _flush_deg(s, deg_sh, zdeg_v, deg2_out)


def _sc_layer23_body(p2, p3, src3, dst3, src3b, dst3b, zrows,
                     s2_out, s3_out,
                     src_v, dst_v, rows_v, stage_v, acc_sh):
    c = lax.axis_index("c")
    s = lax.axis_index("s")
    _zero_acc(s, zrows, None, stage_v, None, acc_sh, None)
    plsc.subcore_barrier()

    @pl.when(c == 0)
    def _():
        pltpu.sync_copy(src3.at[s], src_v)
        pltpu.sync_copy(dst3.at[s], dst_v)
        _edge_loop(p2, src_v, dst_v, rows_v, acc_sh, None, None, None)

    @pl.when(c == 1)
    def _():
        pltpu.sync_copy(src3b.at[s], src_v)
        pltpu.sync_copy(dst3b.at[s], dst_v)
        _edge_loop(p3, src_v, dst_v, rows_v, acc_sh, None, None, None)

    plsc.subcore_barrier()

    @pl.when(c == 0)
    def _():
        _flush(s, acc_sh, stage_v, s2_out)

    @pl.when(c == 1)
    def _():
        _flush(s, acc_sh, stage_v, s3_out)


_sc_layer1 = pl.kernel(
    _sc_layer1_body,
    out_type=[
        jax.ShapeDtypeStruct((_N, 128), _f32),
        jax.ShapeDtypeStruct((_N, 128), _f32),
        jax.ShapeDtypeStruct((_N, _DW), _f32),
        jax.ShapeDtypeStruct((_N, _DW), _f32),
    ],
    mesh=_mesh,
    scratch_types=[
        pltpu.VMEM((_NCHUNK, _CHUNK), jnp.int32),
        pltpu.VMEM((_NCHUNK, _CHUNK), jnp.int32),
        pltpu.VMEM((_NCHUNK, _CHUNK), jnp.int32),
        pltpu.VMEM((_CHUNK, 128), _f32),
        pltpu.VMEM((_CHUNK, _DW), _f32),
        pltpu.VMEM((_ZROWS, 128), _f32),
        pltpu.VMEM((_RPT, _DW), _f32),
        pltpu.VMEM_SHARED((_N, 128), _f32),
        pltpu.VMEM_SHARED((_N, _DW), _f32),
    ],
)

_sc_layer23 = pl.kernel(
    _sc_layer23_body,
    out_type=[
        jax.ShapeDtypeStruct((_N, 128), _f32),
        jax.ShapeDtypeStruct((_N, 128), _f32),
    ],
    mesh=_mesh,
    scratch_types=[
        pltpu.VMEM((_NCHUNK, _CHUNK), jnp.int32),
        pltpu.VMEM((_NCHUNK, _CHUNK), jnp.int32),
        pltpu.VMEM((_CHUNK, 128), _f32),
        pltpu.VMEM((_ZROWS, 128), _f32),
        pltpu.VMEM_SHARED((_N, 128), _f32),
    ],
)


_BLK = 1000


def _tc1_body(s1lo_ref, s1hi_ref, deg_ref, x_ref,
              wl1lo_ref, wl1hi_ref, wr1_ref, b1_ref,
              wl2_ref, wl3_ref, wr2_ref, wr3_ref,
              p2_ref, p3_ref, r2_ref, r3_ref):
    inv = 1.0 / jnp.maximum(deg_ref[:, 0:1], 1.0)
    h = (jnp.dot(s1lo_ref[...] * inv, wl1lo_ref[...],
                 preferred_element_type=_f32)
         + jnp.dot(s1hi_ref[...] * inv, wl1hi_ref[...],
                   preferred_element_type=_f32)
         + jnp.dot(x_ref[...], wr1_ref[...], preferred_element_type=_f32)
         + b1_ref[...])
    emb = jnp.maximum(h, 0.0)
    p2_ref[...] = jnp.dot(emb, wl2_ref[...], preferred_element_type=_f32)
    p3_ref[...] = jnp.dot(emb, wl3_ref[...], preferred_element_type=_f32)
    r2_ref[...] = jnp.dot(emb, wr2_ref[...], preferred_element_type=_f32)
    r3_ref[...] = jnp.dot(emb, wr3_ref[...], preferred_element_type=_f32)


_tc1 = pl.pallas_call(
    _tc1_body,
    grid=(_N // _BLK,),
    in_specs=[
        pl.BlockSpec((_BLK, 128), lambda i: (i, 0)),
        pl.BlockSpec((_BLK, 128), lambda i: (i, 0)),
        pl.BlockSpec((_BLK, _DW), lambda i: (i, 0)),
        pl.BlockSpec((_BLK, 256), lambda i: (i, 0)),
        pl.BlockSpec((128, 256), lambda i: (0, 0)),
        pl.BlockSpec((128, 256), lambda i: (0, 0)),
        pl.BlockSpec((256, 256), lambda i: (0, 0)),
        pl.BlockSpec((256,), lambda i: (0,)),
        pl.BlockSpec((256, 128), lambda i: (0, 0)),
        pl.BlockSpec((256, 128), lambda i: (0, 0)),
        pl.BlockSpec((256, 128), lambda i: (0, 0)),
        pl.BlockSpec((256, 128), lambda i: (0, 0)),
    ],
    out_specs=[pl.BlockSpec((_BLK, 128), lambda i: (i, 0))] * 4,
    out_shape=[jax.ShapeDtypeStruct((_N, 128), _f32)] * 4,
)


def _tc2_body(s2_ref, s3_ref, deg_ref, deg2_ref, r2_ref, r3_ref,
              b2_ref, b3_ref,
              logits_ref, logits2_ref, pred_ref):
    inv = 1.0 / jnp.maximum(deg_ref[:, 0:1], 1.0)
    x1 = s2_ref[...] * inv + r2_ref[...] + b2_ref[...]
    m1 = jnp.max(x1, axis=1, keepdims=True)
    e1 = jnp.exp(x1 - m1)
    logits_ref[...] = e1 / jnp.sum(e1, axis=1, keepdims=True)
    col = lax.broadcasted_iota(jnp.int32, x1.shape, 1)
    pred_ref[...] = jnp.min(jnp.where(x1 == m1, col, x1.shape[1]), axis=1)

    inv2 = 1.0 / jnp.maximum(deg2_ref[:, 0:1], 1.0)
    x2 = s3_ref[...] * inv2 + r3_ref[...] + b3_ref[...]
    m2 = jnp.max(x2, axis=1, keepdims=True)
    e2 = jnp.exp(x2 - m2)
    logits2_ref[...] = e2 / jnp.sum(e2, axis=1, keepdims=True)


_tc2 = pl.pallas_call(
    _tc2_body,
    grid=(_N // _BLK,),
    in_specs=[
        pl.BlockSpec((_BLK, 128), lambda i: (i, 0)),
        pl.BlockSpec((_BLK, 128), lambda i: (i, 0)),
        pl.BlockSpec((_BLK, _DW), lambda i: (i, 0)),
        pl.BlockSpec((_BLK, _DW), lambda i: (i, 0)),
        pl.BlockSpec((_BLK, 128), lambda i: (i, 0)),
        pl.BlockSpec((_BLK, 128), lambda i: (i, 0)),
        pl.BlockSpec((128,), lambda i: (0,)),
        pl.BlockSpec((128,), lambda i: (0,)),
    ],
    out_specs=[
        pl.BlockSpec((_BLK, 128), lambda i: (i, 0)),
        pl.BlockSpec((_BLK, 128), lambda i: (i, 0)),
        pl.BlockSpec((_BLK,), lambda i: (i,)),
    ],
    out_shape=[
        jax.ShapeDtypeStruct((_N, 128), _f32),
        jax.ShapeDtypeStruct((_N, 128), _f32),
        jax.ShapeDtypeStruct((_N,), jnp.int32),
    ],
)


def kernel(x, edge_index, edge_index_2,
           W_l1, W_r1, b1, W_l2, W_r2, b2, W_l3, W_r3, b3):
    x_lo = x[:, :128]
    x_hi = x[:, 128:]
    src3 = edge_index[0].reshape(_NTILES, _NCHUNK, _CHUNK)
    dst3 = edge_index[1].reshape(_NTILES, _NCHUNK, _CHUNK)
    src3b = edge_index_2[0].reshape(_NTILES, _NCHUNK, _CHUNK)
    dst3b = edge_index_2[1].reshape(_NTILES, _NCHUNK, _CHUNK)
    zrows = jnp.zeros((_ZROWS, 128), _f32)
    zdeg = jnp.zeros((_RPT, _DW), _f32)
    ones = jnp.ones((_CHUNK, _DW), _f32)

    s1lo, s1hi, deg, deg2 = _sc_layer1(
        x_lo, x_hi, src3, dst3, dst3b, zrows, zdeg, ones)
    p2, p3, r2, r3 = _tc1(
        s1lo, s1hi, deg, x, W_l1[:128], W_l1[128:], W_r1, b1,
        W_l2, W_l3, W_r2, W_r3)
    s2, s3 = _sc_layer23(p2, p3, src3, dst3, src3b, dst3b, zrows)
    logits, logits2, pred = _tc2(s2, s3, deg, deg2, r2, r3, b2, b3)
    return (logits, logits2, pred)


# speedup vs baseline: 4.5529x; 4.5529x over previous
import jax
import jax.numpy as jnp
from jax import lax
from jax.experimental import pallas as pl
from jax.experimental.pallas import tpu as pltpu
from jax.experimental.pallas import tpu_sc as plsc

_N = 10000
_E = 160000
_NTILES = 16
_NCORES = 2
_NP = 10240
_CHUNK = 80
_NCHUNK = 125
_RPT = _NP // _NTILES
_ZROWS = 128
_NZ = _RPT // _ZROWS

_mesh = plsc.VectorSubcoreMesh(
    core_axis_name="c", subcore_axis_name="s",
    num_cores=_NCORES, num_subcores=_NTILES)

_f32 = jnp.float32


def _zero_acc(s, zrows, zdeg, stage_v, zdeg_v, acc_sh, deg_sh):
    r0 = s * _RPT
    pltpu.sync_copy(zrows, stage_v)
    for i in range(_NZ):
        pltpu.sync_copy(stage_v, acc_sh.at[pl.ds(r0 + i * _ZROWS, _ZROWS)])
    if zdeg is not None:
        pltpu.sync_copy(zdeg, zdeg_v)
        pltpu.sync_copy(zdeg_v, deg_sh.at[pl.ds(r0, _RPT)])


def _edge_loop(table, src_v, dst_v, rows_v, acc_sh):
    def body(j, carry):
        pltpu.sync_copy(table.at[src_v.at[j]], rows_v)
        pltpu.sync_copy(rows_v, acc_sh.at[dst_v.at[j]], add=True)
        return carry
    lax.fori_loop(0, _NCHUNK, body, 0)


def _flush(s, acc_sh, stage_v, out):
    r0 = s * _RPT
    for i in range(_NZ):
        pltpu.sync_copy(acc_sh.at[pl.ds(r0 + i * _ZROWS, _ZROWS)], stage_v)
        pltpu.sync_copy(stage_v, out.at[pl.ds(r0 + i * _ZROWS, _ZROWS)])


def _flush_deg(s, deg_sh, zdeg_v, out):
    r0 = s * _RPT
    pltpu.sync_copy(deg_sh.at[pl.ds(r0, _RPT)], zdeg_v)
    pltpu.sync_copy(zdeg_v, out.at[pl.ds(r0, _RPT)])


def _sc_deg_body(dst3, dst3b, zdeg, ones,
                 deg_out, deg2_out,
                 dst_v, ones_v, zdeg_v, deg_sh):
    c = lax.axis_index("c")
    s = lax.axis_index("s")
    r0 = s * _RPT
    pltpu.sync_copy(zdeg, zdeg_v)
    pltpu.sync_copy(zdeg_v, deg_sh.at[pl.ds(r0, _RPT)])
    plsc.subcore_barrier()
    pltpu.sync_copy(ones, ones_v)

    @pl.when(c == 0)
    def _():
        pltpu.sync_copy(dst3.at[s], dst_v)

    @pl.when(c == 1)
    def _():
        pltpu.sync_copy(dst3b.at[s], dst_v)

    def body(j, carry):
        pltpu.sync_copy(ones_v, deg_sh.at[dst_v.at[j]], add=True)
        return carry
    lax.fori_loop(0, _NCHUNK, body, 0)
    plsc.subcore_barrier()
    pltpu.sync_copy(deg_sh.at[pl.ds(r0, _RPT)], zdeg_v)

    @pl.when(c == 0)
    def _():
        pltpu.sync_copy(zdeg_v, deg_out.at[pl.ds(r0, _RPT)])

    @pl.when(c == 1)
    def _():
        pltpu.sync_copy(zdeg_v, deg2_out.at[pl.ds(r0, _RPT)])


def _sc_agg_body(tlo, thi, src3, dst3, zrows,
                 slo_out, shi_out,
                 src_v, dst_v, rows_v, stage_v, acc_sh):
    c = lax.axis_index("c")
    s = lax.axis_index("s")
    _zero_acc(s, zrows, None, stage_v, None, acc_sh, None)
    plsc.subcore_barrier()
    pltpu.sync_copy(src3.at[s], src_v)
    pltpu.sync_copy(dst3.at[s], dst_v)

    @pl.when(c == 0)
    def _():
        _edge_loop(tlo, src_v, dst_v, rows_v, acc_sh)

    @pl.when(c == 1)
    def _():
        _edge_loop(thi, src_v, dst_v, rows_v, acc_sh)

    plsc.subcore_barrier()

    @pl.when(c == 0)
    def _():
        _flush(s, acc_sh, stage_v, slo_out)

    @pl.when(c == 1)
    def _():
        _flush(s, acc_sh, stage_v, shi_out)


_sc_deg = pl.kernel(
    _sc_deg_body,
    out_type=[
        jax.ShapeDtypeStruct((_NP,), _f32),
        jax.ShapeDtypeStruct((_NP,), _f32),
    ],
    mesh=_mesh,
    compiler_params=pltpu.CompilerParams(use_tc_tiling_on_sc=False),
    scratch_types=[
        pltpu.VMEM((_NCHUNK, _CHUNK), jnp.int32),
        pltpu.VMEM((_CHUNK,), _f32),
        pltpu.VMEM((_RPT,), _f32),
        pltpu.VMEM_SHARED((_NP,), _f32),
    ],
)

_sc_agg = pl.kernel(
    _sc_agg_body,
    out_type=[
        jax.ShapeDtypeStruct((_NP, 128), _f32),
        jax.ShapeDtypeStruct((_NP, 128), _f32),
    ],
    mesh=_mesh,
    compiler_params=pltpu.CompilerParams(use_tc_tiling_on_sc=False),
    scratch_types=[
        pltpu.VMEM((_NCHUNK, _CHUNK), jnp.int32),
        pltpu.VMEM((_NCHUNK, _CHUNK), jnp.int32),
        pltpu.VMEM((_CHUNK, 128), _f32),
        pltpu.VMEM((_ZROWS, 128), _f32),
        pltpu.VMEM_SHARED((_NP, 128), _f32),
    ],
)


def _sc23_body(elo, ehi, src3, dst3, src3b, dst3b, zrows,
               s2lo_out, s2hi_out, s3lo_out, s3hi_out,
               src_v, dst_v, rows_v, stage_v, acc_sh):
    c = lax.axis_index("c")
    s = lax.axis_index("s")
    _zero_acc(s, zrows, None, stage_v, None, acc_sh, None)
    plsc.subcore_barrier()
    pltpu.sync_copy(src3.at[s], src_v)
    pltpu.sync_copy(dst3.at[s], dst_v)

    @pl.when(c == 0)
    def _():
        _edge_loop(elo, src_v, dst_v, rows_v, acc_sh)

    @pl.when(c == 1)
    def _():
        _edge_loop(ehi, src_v, dst_v, rows_v, acc_sh)

    plsc.subcore_barrier()

    @pl.when(c == 0)
    def _():
        _flush(s, acc_sh, stage_v, s2lo_out)

    @pl.when(c == 1)
    def _():
        _flush(s, acc_sh, stage_v, s2hi_out)

    _zero_acc(s, zrows, None, stage_v, None, acc_sh, None)
    pltpu.sync_copy(src3b.at[s], src_v)
    pltpu.sync_copy(dst3b.at[s], dst_v)
    plsc.subcore_barrier()

    @pl.when(c == 0)
    def _():
        _edge_loop(elo, src_v, dst_v, rows_v, acc_sh)

    @pl.when(c == 1)
    def _():
        _edge_loop(ehi, src_v, dst_v, rows_v, acc_sh)

    plsc.subcore_barrier()

    @pl.when(c == 0)
    def _():
        _flush(s, acc_sh, stage_v, s3lo_out)

    @pl.when(c == 1)
    def _():
        _flush(s, acc_sh, stage_v, s3hi_out)


_sc23 = pl.kernel(
    _sc23_body,
    out_type=[jax.ShapeDtypeStruct((_NP, 128), _f32)] * 4,
    mesh=_mesh,
    compiler_params=pltpu.CompilerParams(use_tc_tiling_on_sc=False),
    scratch_types=[
        pltpu.VMEM((_NCHUNK, _CHUNK), jnp.int32),
        pltpu.VMEM((_NCHUNK, _CHUNK), jnp.int32),
        pltpu.VMEM((_CHUNK, 128), _f32),
        pltpu.VMEM((_ZROWS, 128), _f32),
        pltpu.VMEM_SHARED((_NP, 128), _f32),
    ],
)


_BLK = 1024


def _tc1_body(s1lo_ref, s1hi_ref, deg_ref, x_ref,
              wl1_ref, wr1_ref, b1_ref,
              elo_ref, ehi_ref):
    s1 = jnp.concatenate([s1lo_ref[...], s1hi_ref[...]], axis=1)
    agg = s1 / jnp.maximum(deg_ref[...], 1.0)
    h = (jnp.dot(agg, wl1_ref[...], preferred_element_type=_f32)
         + b1_ref[...]
         + jnp.dot(x_ref[...], wr1_ref[...], preferred_element_type=_f32))
    emb = jnp.maximum(h, 0.0)
    elo_ref[...] = emb[:, :128]
    ehi_ref[...] = emb[:, 128:]


_tc1 = pl.pallas_call(
    _tc1_body,
    grid=(_NP // _BLK,),
    in_specs=[
        pl.BlockSpec((_BLK, 128), lambda i: (i, 0)),
        pl.BlockSpec((_BLK, 128), lambda i: (i, 0)),
        pl.BlockSpec((_BLK, 1), lambda i: (i, 0)),
        pl.BlockSpec((_BLK, 256), lambda i: (i, 0)),
        pl.BlockSpec((256, 256), lambda i: (0, 0)),
        pl.BlockSpec((256, 256), lambda i: (0, 0)),
        pl.BlockSpec((256,), lambda i: (0,)),
    ],
    out_specs=[pl.BlockSpec((_BLK, 128), lambda i: (i, 0))] * 2,
    out_shape=[jax.ShapeDtypeStruct((_NP, 128), _f32)] * 2,
)


def _tc2_body(s2lo_ref, s2hi_ref, s3lo_ref, s3hi_ref, deg_ref, deg2_ref,
              elo_ref, ehi_ref, wl2_ref, wr2_ref, b2_ref,
              wl3_ref, wr3_ref, b3_ref,
              logits_ref, logits2_ref, pred_ref):
    emb = jnp.concatenate([elo_ref[...], ehi_ref[...]], axis=1)

    agg2 = (jnp.concatenate([s2lo_ref[...], s2hi_ref[...]], axis=1)
            / jnp.maximum(deg_ref[...], 1.0))
    x1 = (jnp.dot(agg2, wl2_ref[...], preferred_element_type=_f32)
          + b2_ref[...]
          + jnp.dot(emb, wr2_ref[...], preferred_element_type=_f32))
    m1 = jnp.max(x1, axis=1, keepdims=True)
    e1 = jnp.exp(x1 - m1)
    logits_ref[...] = e1 / jnp.sum(e1, axis=1, keepdims=True)
    col = lax.broadcasted_iota(jnp.int32, x1.shape, 1)
    i = pl.program_id(0)
    pred_ref[pl.ds(i * _BLK, _BLK)] = jnp.min(
        jnp.where(x1 == m1, col, x1.shape[1]), axis=1)

    agg3 = (jnp.concatenate([s3lo_ref[...], s3hi_ref[...]], axis=1)
            / jnp.maximum(deg2_ref[...], 1.0))
    x2 = (jnp.dot(agg3, wl3_ref[...], preferred_element_type=_f32)
          + b3_ref[...]
          + jnp.dot(emb, wr3_ref[...], preferred_element_type=_f32))
    m2 = jnp.max(x2, axis=1, keepdims=True)
    e2 = jnp.exp(x2 - m2)
    logits2_ref[...] = e2 / jnp.sum(e2, axis=1, keepdims=True)


_tc2 = pl.pallas_call(
    _tc2_body,
    grid=(_NP // _BLK,),
    in_specs=[
        pl.BlockSpec((_BLK, 128), lambda i: (i, 0)),
        pl.BlockSpec((_BLK, 128), lambda i: (i, 0)),
        pl.BlockSpec((_BLK, 128), lambda i: (i, 0)),
        pl.BlockSpec((_BLK, 128), lambda i: (i, 0)),
        pl.BlockSpec((_BLK, 1), lambda i: (i, 0)),
        pl.BlockSpec((_BLK, 1), lambda i: (i, 0)),
        pl.BlockSpec((_BLK, 128), lambda i: (i, 0)),
        pl.BlockSpec((_BLK, 128), lambda i: (i, 0)),
        pl.BlockSpec((256, 128), lambda i: (0, 0)),
        pl.BlockSpec((256, 128), lambda i: (0, 0)),
        pl.BlockSpec((128,), lambda i: (0,)),
        pl.BlockSpec((256, 128), lambda i: (0, 0)),
        pl.BlockSpec((256, 128), lambda i: (0, 0)),
        pl.BlockSpec((128,), lambda i: (0,)),
    ],
    out_specs=[
        pl.BlockSpec((_BLK, 128), lambda i: (i, 0)),
        pl.BlockSpec((_BLK, 128), lambda i: (i, 0)),
        pl.BlockSpec((_NP,), lambda i: (0,)),
    ],
    out_shape=[
        jax.ShapeDtypeStruct((_NP, 128), _f32),
        jax.ShapeDtypeStruct((_NP, 128), _f32),
        jax.ShapeDtypeStruct((_NP,), jnp.int32),
    ],
)


def kernel(x, edge_index, edge_index_2,
           W_l1, W_r1, b1, W_l2, W_r2, b2, W_l3, W_r3, b3):
    x_lo = x[:, :128]
    x_hi = x[:, 128:]
    src3 = edge_index[0].reshape(_NTILES, _NCHUNK, _CHUNK)
    dst3 = edge_index[1].reshape(_NTILES, _NCHUNK, _CHUNK)
    src3b = edge_index_2[0].reshape(_NTILES, _NCHUNK, _CHUNK)
    dst3b = edge_index_2[1].reshape(_NTILES, _NCHUNK, _CHUNK)
    zrows = jnp.zeros((_ZROWS, 128), _f32)
    zdeg = jnp.zeros((_RPT,), _f32)
    ones = jnp.ones((_CHUNK,), _f32)
    xp = jnp.pad(x, ((0, _NP - _N), (0, 0)))

    deg, deg2 = _sc_deg(dst3, dst3b, zdeg, ones)
    deg = deg.reshape(_NP, 1)
    deg2 = deg2.reshape(_NP, 1)
    s1lo, s1hi = _sc_agg(x_lo, x_hi, src3, dst3, zrows)
    elo, ehi = _tc1(s1lo, s1hi, deg, xp, W_l1, W_r1, b1)
    s2lo, s2hi, s3lo, s3hi = _sc23(
        elo, ehi, src3, dst3, src3b, dst3b, zrows)
    logits, logits2, pred = _tc2(
        s2lo, s2hi, s3lo, s3hi, deg, deg2, elo, ehi,
        W_l2, W_r2, b2, W_l3, W_r3, b3)
    return (logits[:_N], logits2[:_N], pred[:_N])
